# Initial kernel scaffold; baseline (speedup 1.0000x reference)
import jax
import jax.numpy as jnp
from jax.experimental import pallas as pl


def kernel(x, z_vis2048, W_sem, W_temp, b_temp, W_a1, b_a1, W_a2, b_a2):
    raise NotImplementedError("write your pallas kernel here")


# speedup vs baseline: 12.4303x; 12.4303x over previous
import functools

import jax
import jax.numpy as jnp
from jax import lax
from jax.experimental import pallas as pl

_K = 12


def _sem_proj_body(z_ref, ws_ref, o_ref):
    o_ref[...] = lax.dot_general(
        z_ref[...], ws_ref[...], (((1,), (1,)), ((), ())),
        preferred_element_type=jnp.float32)


def _fused_body(w_ref, x_ref, wt_ref, bt_ref, wa1_ref, ba1_ref, wa2_ref,
                z_ref, as_ref):
    w = w_ref[0]
    logits = lax.dot_general(
        w, w, (((1,), (1,)), ((), ())),
        preferred_element_type=jnp.float32) * 0.25
    m = jnp.max(logits, axis=1, keepdims=True)
    e = jnp.exp(logits - m)
    p = e / jnp.sum(e, axis=1, keepdims=True)

    cur = p
    thr = None
    for _ in range(_K):
        thr = jnp.max(cur, axis=1, keepdims=True)
        cur = jnp.where(cur >= thr, -1.0, cur)
    masked = jnp.where(p >= thr, p, 0.0)
    a_sym = (masked + masked.T) * 0.5
    as_ref[0] = a_sym

    h1 = lax.dot_general(
        x_ref[0], wt_ref[...], (((1,), (1,)), ((), ())),
        preferred_element_type=jnp.float32) + bt_ref[...]
    h2 = jnp.dot(a_sym, h1, preferred_element_type=jnp.float32)
    a1 = jnp.tanh(
        lax.dot_general(h2, wa1_ref[...], (((1,), (1,)), ((), ())),
                        preferred_element_type=jnp.float32) + ba1_ref[...])
    s = jnp.sum(a1 * wa2_ref[...], axis=1, keepdims=True)
    es = jnp.exp(s - jnp.max(s, axis=0, keepdims=True))
    score = es / jnp.sum(es, axis=0, keepdims=True)
    z_ref[0, 0] = jnp.sum(score * h2, axis=0)


def kernel(x, z_vis2048, W_sem, W_temp, b_temp, W_a1, b_a1, W_a2, b_a2):
    del b_a2
    B, C, T = x.shape
    D = W_temp.shape[0]
    F = W_sem.shape[0] // C
    N = W_sem.shape[0]
    NT = 512

    w_flat = pl.pallas_call(
        _sem_proj_body,
        grid=(pl.cdiv(N, NT),),
        in_specs=[
            pl.BlockSpec((B, W_sem.shape[1]), lambda i: (0, 0)),
            pl.BlockSpec((NT, W_sem.shape[1]), lambda i: (i, 0)),
        ],
        out_specs=pl.BlockSpec((B, NT), lambda i: (0, i)),
        out_shape=jax.ShapeDtypeStruct((B, N), jnp.float32),
    )(z_vis2048, W_sem)
    w3 = w_flat.reshape(B, C, F)

    z_graph, a_sym = pl.pallas_call(
        _fused_body,
        grid=(B,),
        in_specs=[
            pl.BlockSpec((1, C, F), lambda b: (b, 0, 0)),
            pl.BlockSpec((1, C, T), lambda b: (b, 0, 0)),
            pl.BlockSpec((D, T), lambda b: (0, 0)),
            pl.BlockSpec((1, D), lambda b: (0, 0)),
            pl.BlockSpec((D // 2, D), lambda b: (0, 0)),
            pl.BlockSpec((1, D // 2), lambda b: (0, 0)),
            pl.BlockSpec((1, D // 2), lambda b: (0, 0)),
        ],
        out_specs=[
            pl.BlockSpec((1, 1, D), lambda b: (b, 0, 0)),
            pl.BlockSpec((1, C, C), lambda b: (b, 0, 0)),
        ],
        out_shape=[
            jax.ShapeDtypeStruct((B, 1, D), jnp.float32),
            jax.ShapeDtypeStruct((B, C, C), jnp.float32),
        ],
    )(w3, x, W_temp, b_temp.reshape(1, D), W_a1, b_a1.reshape(1, D // 2),
      W_a2)
    return (z_graph.reshape(B, D), a_sym)


# speedup vs baseline: 14.2167x; 1.1437x over previous
import functools

import jax
import jax.numpy as jnp
from jax import lax
from jax.experimental import pallas as pl

_K = 12
_BB = 2
_NEG = -3.0e38


def _sem_proj_body(z_ref, ws_ref, o_ref):
    o_ref[...] = lax.dot_general(
        z_ref[...], ws_ref[...], (((1,), (1,)), ((), ())),
        preferred_element_type=jnp.float32)


def _fused_body(w_ref, wt_ref, x_ref, wtemp_ref, bt_ref, wa1_ref, ba1_ref,
                wa2_ref, z_ref, as_ref):
    for i in range(_BB):
        logits = lax.dot_general(
            w_ref[i], wt_ref[i], (((1,), (0,)), ((), ())),
            preferred_element_type=jnp.float32) * 0.25

        cur = logits
        for _ in range(_K - 1):
            t = jnp.max(cur, axis=1, keepdims=True)
            cur = jnp.where(cur >= t, _NEG, cur)
        thr = jnp.max(cur, axis=1, keepdims=True)

        m = jnp.max(logits, axis=1, keepdims=True)
        e = jnp.exp(logits - m)
        p = e / jnp.sum(e, axis=1, keepdims=True)
        masked = jnp.where(logits >= thr, p, 0.0)
        a_sym = (masked + masked.T) * 0.5
        as_ref[i] = a_sym

        h1 = lax.dot_general(
            x_ref[i], wtemp_ref[...], (((1,), (1,)), ((), ())),
            preferred_element_type=jnp.float32) + bt_ref[...]
        h2 = jnp.dot(a_sym, h1, preferred_element_type=jnp.float32)
        a1 = jnp.tanh(
            lax.dot_general(h2, wa1_ref[...], (((1,), (1,)), ((), ())),
                            preferred_element_type=jnp.float32) + ba1_ref[...])
        s = jnp.sum(a1 * wa2_ref[...], axis=1, keepdims=True)
        es = jnp.exp(s - jnp.max(s, axis=0, keepdims=True))
        score = es / jnp.sum(es, axis=0, keepdims=True)
        z_ref[i, 0] = jnp.sum(score * h2, axis=0)


def kernel(x, z_vis2048, W_sem, W_temp, b_temp, W_a1, b_a1, W_a2, b_a2):
    del b_a2
    B, C, T = x.shape
    D = W_temp.shape[0]
    F = W_sem.shape[0] // C
    N = W_sem.shape[0]
    NT = 512

    w_flat = pl.pallas_call(
        _sem_proj_body,
        grid=(pl.cdiv(N, NT),),
        in_specs=[
            pl.BlockSpec((B, W_sem.shape[1]), lambda i: (0, 0)),
            pl.BlockSpec((NT, W_sem.shape[1]), lambda i: (i, 0)),
        ],
        out_specs=pl.BlockSpec((B, NT), lambda i: (0, i)),
        out_shape=jax.ShapeDtypeStruct((B, N), jnp.float32),
    )(z_vis2048, W_sem)
    w3 = w_flat.reshape(B, C, F)
    w3t = jnp.swapaxes(w3, 1, 2)

    z_graph, a_sym = pl.pallas_call(
        _fused_body,
        grid=(B // _BB,),
        in_specs=[
            pl.BlockSpec((_BB, C, F), lambda b: (b, 0, 0)),
            pl.BlockSpec((_BB, F, C), lambda b: (b, 0, 0)),
            pl.BlockSpec((_BB, C, T), lambda b: (b, 0, 0)),
            pl.BlockSpec((D, T), lambda b: (0, 0)),
            pl.BlockSpec((1, D), lambda b: (0, 0)),
            pl.BlockSpec((D // 2, D), lambda b: (0, 0)),
            pl.BlockSpec((1, D // 2), lambda b: (0, 0)),
            pl.BlockSpec((1, D // 2), lambda b: (0, 0)),
        ],
        out_specs=[
            pl.BlockSpec((_BB, 1, D), lambda b: (b, 0, 0)),
            pl.BlockSpec((_BB, C, C), lambda b: (b, 0, 0)),
        ],
        out_shape=[
            jax.ShapeDtypeStruct((B, 1, D), jnp.float32),
            jax.ShapeDtypeStruct((B, C, C), jnp.float32),
        ],
    )(w3, w3t, x, W_temp, b_temp.reshape(1, D), W_a1,
      b_a1.reshape(1, D // 2), W_a2)
    return (z_graph.reshape(B, D), a_sym)


# speedup vs baseline: 15.5690x; 1.0951x over previous
import functools

import jax
import jax.numpy as jnp
from jax import lax
from jax.experimental import pallas as pl

_K = 12
_BB = 4
_NEG = -3.0e38


def _sem_proj_body(z_ref, ws_ref, o_ref):
    o_ref[...] = lax.dot_general(
        z_ref[...], ws_ref[...], (((1,), (1,)), ((), ())),
        preferred_element_type=jnp.float32)


def _fused_body(w_ref, wt_ref, x_ref, wtemp_ref, bt_ref, wa1_ref, ba1_ref,
                wa2_ref, z_ref, as_ref):
    for i in range(_BB):
        logits = lax.dot_general(
            w_ref[i], wt_ref[i], (((1,), (0,)), ((), ())),
            preferred_element_type=jnp.float32) * 0.25

        m = jnp.max(logits, axis=1, keepdims=True)
        cur = jnp.where(logits >= m, _NEG, logits)
        for _ in range(_K - 2):
            t = jnp.max(cur, axis=1, keepdims=True)
            cur = jnp.where(cur >= t, _NEG, cur)
        thr = jnp.max(cur, axis=1, keepdims=True)

        e = jnp.exp(logits - m)
        rs = 0.5 / jnp.sum(e, axis=1, keepdims=True)
        mh = jnp.where(logits >= thr, e * rs, 0.0)
        a_sym = mh + mh.T
        as_ref[i] = a_sym

        h1 = lax.dot_general(
            x_ref[i], wtemp_ref[...], (((1,), (1,)), ((), ())),
            preferred_element_type=jnp.float32) + bt_ref[...]
        h2 = jnp.dot(a_sym, h1, preferred_element_type=jnp.float32)
        a1 = jnp.tanh(
            lax.dot_general(h2, wa1_ref[...], (((1,), (1,)), ((), ())),
                            preferred_element_type=jnp.float32) + ba1_ref[...])
        s = jnp.sum(a1 * wa2_ref[...], axis=1, keepdims=True)
        es = jnp.exp(s - jnp.max(s, axis=0, keepdims=True))
        score = es / jnp.sum(es, axis=0, keepdims=True)
        z_ref[i, 0] = jnp.sum(score * h2, axis=0)


def kernel(x, z_vis2048, W_sem, W_temp, b_temp, W_a1, b_a1, W_a2, b_a2):
    del b_a2
    B, C, T = x.shape
    D = W_temp.shape[0]
    F = W_sem.shape[0] // C
    N = W_sem.shape[0]
    NT = 512

    w_flat = pl.pallas_call(
        _sem_proj_body,
        grid=(pl.cdiv(N, NT),),
        in_specs=[
            pl.BlockSpec((B, W_sem.shape[1]), lambda i: (0, 0)),
            pl.BlockSpec((NT, W_sem.shape[1]), lambda i: (i, 0)),
        ],
        out_specs=pl.BlockSpec((B, NT), lambda i: (0, i)),
        out_shape=jax.ShapeDtypeStruct((B, N), jnp.float32),
    )(z_vis2048, W_sem)
    w3 = w_flat.reshape(B, C, F)
    w3t = jnp.swapaxes(w3, 1, 2)

    z_graph, a_sym = pl.pallas_call(
        _fused_body,
        grid=(B // _BB,),
        in_specs=[
            pl.BlockSpec((_BB, C, F), lambda b: (b, 0, 0)),
            pl.BlockSpec((_BB, F, C), lambda b: (b, 0, 0)),
            pl.BlockSpec((_BB, C, T), lambda b: (b, 0, 0)),
            pl.BlockSpec((D, T), lambda b: (0, 0)),
            pl.BlockSpec((1, D), lambda b: (0, 0)),
            pl.BlockSpec((D // 2, D), lambda b: (0, 0)),
            pl.BlockSpec((1, D // 2), lambda b: (0, 0)),
            pl.BlockSpec((1, D // 2), lambda b: (0, 0)),
        ],
        out_specs=[
            pl.BlockSpec((_BB, 1, D), lambda b: (b, 0, 0)),
            pl.BlockSpec((_BB, C, C), lambda b: (b, 0, 0)),
        ],
        out_shape=[
            jax.ShapeDtypeStruct((B, 1, D), jnp.float32),
            jax.ShapeDtypeStruct((B, C, C), jnp.float32),
        ],
    )(w3, w3t, x, W_temp, b_temp.reshape(1, D), W_a1,
      b_a1.reshape(1, D // 2), W_a2)
    return (z_graph.reshape(B, D), a_sym)
